# baseline (speedup 1.0000x reference)
import functools

import jax
import jax.numpy as jnp
from jax import lax
from jax.experimental import pallas as pl
from jax.experimental.pallas import tpu as pltpu
from jax.experimental.pallas import tpu_sc as plsc

_B, _S, _T, _R = 4, 2048, 3, 256
_ALPHA, _MEM = 16, 16
_PAIRS = _B * _S
_V2 = _R * _ALPHA ** 2
_V3 = _R * _ALPHA ** 3

_NC, _NS = 2, 16
_NW = _NC * _NS
_PPW = _PAIRS // _NW

_G = 4
_NBUF = 2
_NGW = _PPW // _G


def _presum_body(w_ref, o_ref):
    x = w_ref[0] + w_ref[1]
    r = lax.broadcasted_iota(jnp.int32, (16, 16), 0)
    c = lax.broadcasted_iota(jnp.int32, (16, 16), 1)
    eye = (r == c).astype(jnp.float32)
    o_ref[...] = lax.dot_general(
        x, eye, dimension_numbers=(((0,), (0,)), ((), ())),
        preferred_element_type=jnp.float32)


def _presum(w, v, chunk):
    wt = jnp.transpose(w, (0, 2, 1))
    return pl.pallas_call(
        _presum_body,
        grid=(v // chunk,),
        in_specs=[pl.BlockSpec((2, 16, chunk), lambda i: (0, 0, i))],
        out_specs=pl.BlockSpec((chunk, 16), lambda i: (i, 0)),
        out_shape=jax.ShapeDtypeStruct((v, 16), jnp.float32),
    )(wt)


def _idx_body(c_ref, o_ref):
    x = c_ref[0]
    c0 = x[0]
    c1 = x[1]
    c2 = x[2]
    r = lax.broadcasted_iota(jnp.int32, c0.shape, 1)
    idx2 = r * 256 + c1 + c2 * 16
    idx3 = r * 4096 + c0 + c1 * 16 + c2 * 256
    cat = jnp.concatenate([idx2, idx3], axis=1)
    o_ref[...] = cat.reshape(o_ref.shape)


def _idx(codes):
    ct = jnp.transpose(codes, (0, 2, 1, 3))
    return pl.pallas_call(
        _idx_body,
        grid=(_B, 2),
        in_specs=[pl.BlockSpec((1, _T, _S // 2, _R), lambda i, j: (i, 0, j, 0))],
        out_specs=pl.BlockSpec((4096, 128), lambda i, j: (i * 2 + j, 0)),
        out_shape=jax.ShapeDtypeStruct((_PAIRS * 4, 128), jnp.int32),
    )(ct)


def _assemble_body(x_ref, o_ref):
    for ct in range(64):
        o_ref[0, :, ct * 128:(ct + 1) * 128] = x_ref[:, ct, :]


def _assemble(out2d):
    x = out2d.reshape(_PAIRS, 64, 128)
    return pl.pallas_call(
        _assemble_body,
        grid=(_B, 8),
        in_specs=[pl.BlockSpec((256, 64, 128), lambda i, j: (i * 8 + j, 0, 0))],
        out_specs=pl.BlockSpec((1, 256, 8192), lambda i, j: (i, j, 0)),
        out_shape=jax.ShapeDtypeStruct((_B, _S, 2 * _R * _MEM), jnp.float32),
    )(x)


def _sc_gather(idx2d, w2s, w3s):
    mesh = plsc.VectorSubcoreMesh(
        core_axis_name="c", subcore_axis_name="s",
        num_cores=_NC, num_subcores=_NS)

    @functools.partial(
        pl.kernel,
        out_type=jax.ShapeDtypeStruct((_PAIRS * 512, _MEM), jnp.float32),
        mesh=mesh,
        scratch_types=[
            pltpu.VMEM((_NBUF, 4 * _G, 128), jnp.int32),
            pltpu.VMEM((_NBUF, 4 * _G * 128, _MEM), jnp.float32),
            pltpu.SemaphoreType.DMA((_NBUF,)),
            pltpu.SemaphoreType.DMA((_NBUF,)),
            pltpu.SemaphoreType.DMA((_NBUF,)),
        ],
        compiler_params=pltpu.CompilerParams(use_tc_tiling_on_sc=False),
    )
    def k(idx_hbm, w2_hbm, w3_hbm, out_hbm, idx_v, rows_v, isem, gsem, ssem):
        wid = lax.axis_index("s") * _NC + lax.axis_index("c")
        g0 = wid * _NGW

        def idx_cp(slot, g):
            return pltpu.make_async_copy(
                idx_hbm.at[pl.ds((g0 + g) * (4 * _G), 4 * _G)],
                idx_v.at[slot], isem.at[slot])

        def gath_cps(slot):
            cps = []
            for q in range(_G):
                for j in range(4):
                    tbl = w2_hbm if j < 2 else w3_hbm
                    cps.append(pltpu.make_async_copy(
                        tbl.at[idx_v.at[slot, 4 * q + j]],
                        rows_v.at[slot, pl.ds((4 * q + j) * 128, 128)],
                        gsem.at[slot]))
            return cps

        def store_cp(slot, g):
            return pltpu.make_async_copy(
                rows_v.at[slot],
                out_hbm.at[pl.ds((g0 + g) * (512 * _G), 512 * _G)], ssem.at[slot])

        idx_cp(0, 0).start()
        idx_cp(1, 1).start()
        idx_cp(0, 0).wait()
        for cp in gath_cps(0):
            cp.start()
        for cp in gath_cps(0):
            cp.wait()
        store_cp(0, 0).start()
        idx_cp(1, 1).wait()
        for cp in gath_cps(1):
            cp.start()
        idx_cp(0, 2).start()

        @pl.loop(2, _NGW - 2, step=_NBUF)
        def _outer(go):
            for b in range(_NBUF):
                g = go + b
                prev = 1 - b
                for cp in gath_cps(prev):
                    cp.wait()
                store_cp(prev, g - 1).start()
                store_cp(b, g - 2).wait()
                idx_cp(b, g).wait()
                for cp in gath_cps(b):
                    cp.start()
                idx_cp(prev, g + 1).start()

        for cp in gath_cps(1):
            cp.wait()
        store_cp(1, _NGW - 3).start()
        store_cp(0, _NGW - 4).wait()
        idx_cp(0, _NGW - 2).wait()
        for cp in gath_cps(0):
            cp.start()
        idx_cp(1, _NGW - 1).start()

        for cp in gath_cps(0):
            cp.wait()
        store_cp(0, _NGW - 2).start()
        store_cp(1, _NGW - 3).wait()
        idx_cp(1, _NGW - 1).wait()
        for cp in gath_cps(1):
            cp.start()

        for cp in gath_cps(1):
            cp.wait()
        store_cp(1, _NGW - 1).start()
        store_cp(0, _NGW - 2).wait()
        store_cp(1, _NGW - 1).wait()

    return k(idx2d, w2s, w3s)


def kernel(route_codes_bstr, W_ngram_2, W_ngram_3):
    idx2d = _idx(route_codes_bstr)
    w2s = _presum(W_ngram_2, _V2, 16384)
    w3s = _presum(W_ngram_3, _V3, 16384)
    out2d = _sc_gather(idx2d, w2s, w3s)
    return _assemble(out2d)


# speedup vs baseline: 1.3368x; 1.3368x over previous
import functools

import jax
import jax.numpy as jnp
from jax import lax
from jax.experimental import pallas as pl
from jax.experimental.pallas import tpu as pltpu
from jax.experimental.pallas import tpu_sc as plsc

_B, _S, _T, _R = 4, 2048, 3, 256
_ALPHA, _MEM = 16, 16
_PAIRS = _B * _S
_V2 = _R * _ALPHA ** 2
_V3 = _R * _ALPHA ** 3

_NC, _NS = 2, 16
_NW = _NC * _NS
_PPW = _PAIRS // _NW

_G = 4
_NBUF = 2
_NGW = _PPW // _G


_CH = 16384
_C8 = _CH // 8


def _presum_body(w_ref, o_ref):
    x = w_ref[0] + w_ref[1]
    r = lax.broadcasted_iota(jnp.int32, (16, 16), 0)
    c = lax.broadcasted_iota(jnp.int32, (16, 16), 1)
    eye = (r == c).astype(jnp.float32)
    for i in range(8):
        xi = x[:, i * _C8:(i + 1) * _C8]
        o_ref[:, i * 16:(i + 1) * 16] = lax.dot_general(
            xi, eye, dimension_numbers=(((0,), (0,)), ((), ())),
            preferred_element_type=jnp.float32)


def _presum(w, v):
    wt = jnp.transpose(w, (0, 2, 1))
    return pl.pallas_call(
        _presum_body,
        grid=(v // _CH,),
        in_specs=[pl.BlockSpec((2, 16, _CH), lambda i: (0, 0, i))],
        out_specs=pl.BlockSpec((_C8, 128), lambda i: (i, 0)),
        out_shape=jax.ShapeDtypeStruct((v // 8, 128), jnp.float32),
    )(wt)


def _perm_idx(t):
    return (t & ~(_CH - 1)) | ((t & (_C8 - 1)) << 3) | ((t >> 11) & 7)


def _idx_body(c_ref, o_ref):
    x = c_ref[0]
    c0 = x[0]
    c1 = x[1]
    c2 = x[2]
    r = lax.broadcasted_iota(jnp.int32, c0.shape, 1)
    idx2 = _perm_idx(r * 256 + c1 + c2 * 16)
    idx3 = _perm_idx(r * 4096 + c0 + c1 * 16 + c2 * 256)
    cat = jnp.concatenate([idx2, idx3], axis=1)
    o_ref[...] = cat.reshape(o_ref.shape)


def _idx(codes):
    ct = jnp.transpose(codes, (0, 2, 1, 3))
    return pl.pallas_call(
        _idx_body,
        grid=(_B, 2),
        in_specs=[pl.BlockSpec((1, _T, _S // 2, _R), lambda i, j: (i, 0, j, 0))],
        out_specs=pl.BlockSpec((4096, 128), lambda i, j: (i * 2 + j, 0)),
        out_shape=jax.ShapeDtypeStruct((_PAIRS * 4, 128), jnp.int32),
    )(ct)


def _assemble_body(x_ref, o_ref):
    for ct in range(64):
        o_ref[0, :, ct * 128:(ct + 1) * 128] = x_ref[:, ct, :]


def _assemble(out2d):
    x = out2d.reshape(_PAIRS, 64, 128)
    return pl.pallas_call(
        _assemble_body,
        grid=(_B, 8),
        in_specs=[pl.BlockSpec((256, 64, 128), lambda i, j: (i * 8 + j, 0, 0))],
        out_specs=pl.BlockSpec((1, 256, 8192), lambda i, j: (i, j, 0)),
        out_shape=jax.ShapeDtypeStruct((_B, _S, 2 * _R * _MEM), jnp.float32),
    )(x)


def _sc_gather(idx2d, w2s, w3s):
    mesh = plsc.VectorSubcoreMesh(
        core_axis_name="c", subcore_axis_name="s",
        num_cores=_NC, num_subcores=_NS)

    @functools.partial(
        pl.kernel,
        out_type=jax.ShapeDtypeStruct((_PAIRS * 512, _MEM), jnp.float32),
        mesh=mesh,
        scratch_types=[
            pltpu.VMEM((_NBUF, 4 * _G, 128), jnp.int32),
            pltpu.VMEM((_NBUF, 4 * _G * 128, _MEM), jnp.float32),
            pltpu.SemaphoreType.DMA((_NBUF,)),
            pltpu.SemaphoreType.DMA((_NBUF,)),
            pltpu.SemaphoreType.DMA((_NBUF,)),
        ],
        compiler_params=pltpu.CompilerParams(use_tc_tiling_on_sc=False),
    )
    def k(idx_hbm, w2_hbm, w3_hbm, out_hbm, idx_v, rows_v, isem, gsem, ssem):
        wid = lax.axis_index("s") * _NC + lax.axis_index("c")
        g0 = wid * _NGW

        def idx_cp(slot, g):
            return pltpu.make_async_copy(
                idx_hbm.at[pl.ds((g0 + g) * (4 * _G), 4 * _G)],
                idx_v.at[slot], isem.at[slot])

        def gath_cps(slot):
            cps = []
            for q in range(_G):
                for j in range(4):
                    tbl = w2_hbm if j < 2 else w3_hbm
                    cps.append(pltpu.make_async_copy(
                        tbl.at[idx_v.at[slot, 4 * q + j]],
                        rows_v.at[slot, pl.ds((4 * q + j) * 128, 128)],
                        gsem.at[slot]))
            return cps

        def store_cp(slot, g):
            return pltpu.make_async_copy(
                rows_v.at[slot],
                out_hbm.at[pl.ds((g0 + g) * (512 * _G), 512 * _G)], ssem.at[slot])

        idx_cp(0, 0).start()
        idx_cp(1, 1).start()
        idx_cp(0, 0).wait()
        for cp in gath_cps(0):
            cp.start()
        for cp in gath_cps(0):
            cp.wait()
        store_cp(0, 0).start()
        idx_cp(1, 1).wait()
        for cp in gath_cps(1):
            cp.start()
        idx_cp(0, 2).start()

        @pl.loop(2, _NGW - 2, step=_NBUF)
        def _outer(go):
            for b in range(_NBUF):
                g = go + b
                prev = 1 - b
                for cp in gath_cps(prev):
                    cp.wait()
                store_cp(prev, g - 1).start()
                store_cp(b, g - 2).wait()
                idx_cp(b, g).wait()
                for cp in gath_cps(b):
                    cp.start()
                idx_cp(prev, g + 1).start()

        for cp in gath_cps(1):
            cp.wait()
        store_cp(1, _NGW - 3).start()
        store_cp(0, _NGW - 4).wait()
        idx_cp(0, _NGW - 2).wait()
        for cp in gath_cps(0):
            cp.start()
        idx_cp(1, _NGW - 1).start()

        for cp in gath_cps(0):
            cp.wait()
        store_cp(0, _NGW - 2).start()
        store_cp(1, _NGW - 3).wait()
        idx_cp(1, _NGW - 1).wait()
        for cp in gath_cps(1):
            cp.start()

        for cp in gath_cps(1):
            cp.wait()
        store_cp(1, _NGW - 1).start()
        store_cp(0, _NGW - 2).wait()
        store_cp(1, _NGW - 1).wait()

    return k(idx2d, w2s, w3s)


def kernel(route_codes_bstr, W_ngram_2, W_ngram_3):
    idx2d = _idx(route_codes_bstr)
    w2s = _presum(W_ngram_2, _V2).reshape(_V2, _MEM)
    w3s = _presum(W_ngram_3, _V3).reshape(_V3, _MEM)
    out2d = _sc_gather(idx2d, w2s, w3s)
    return _assemble(out2d)


# speedup vs baseline: 1.7418x; 1.3029x over previous
import functools

import jax
import jax.numpy as jnp
from jax import lax
from jax.experimental import pallas as pl
from jax.experimental.pallas import tpu as pltpu
from jax.experimental.pallas import tpu_sc as plsc

_B, _S, _T, _R = 4, 2048, 3, 256
_ALPHA, _MEM = 16, 16
_PAIRS = _B * _S
_V2 = _R * _ALPHA ** 2
_V3 = _R * _ALPHA ** 3

_NC, _NS = 2, 16
_NW = _NC * _NS
_PPW = _PAIRS // _NW

_G = 4
_NBUF = 2
_NGW = _PPW // _G


_CH = 16384
_C8 = _CH // 8


def _presum_body(w_ref, o_ref):
    x = w_ref[0] + w_ref[1]
    r = lax.broadcasted_iota(jnp.int32, (128, 128), 0)
    c = lax.broadcasted_iota(jnp.int32, (128, 128), 1)
    eye = (r == c).astype(jnp.float32)
    xs = jnp.concatenate(
        [x[:, i * _C8:(i + 1) * _C8] for i in range(8)], axis=0)
    o_ref[...] = lax.dot_general(
        xs, eye, dimension_numbers=(((0,), (0,)), ((), ())),
        preferred_element_type=jnp.float32)


def _presum(w, v):
    wt = jnp.transpose(w, (0, 2, 1))
    return pl.pallas_call(
        _presum_body,
        grid=(v // _CH,),
        in_specs=[pl.BlockSpec((2, 16, _CH), lambda i: (0, 0, i))],
        out_specs=pl.BlockSpec((_C8, 128), lambda i: (i, 0)),
        out_shape=jax.ShapeDtypeStruct((v // 8, 128), jnp.float32),
    )(wt)


def _perm_idx(t):
    return (t & ~(_CH - 1)) | ((t & (_C8 - 1)) << 3) | ((t >> 11) & 7)


def _idx_body(c_ref, o_ref):
    x = c_ref[0]
    c0 = x[0]
    c1 = x[1]
    c2 = x[2]
    r = lax.broadcasted_iota(jnp.int32, c0.shape, 1)
    idx2 = _perm_idx(r * 256 + c1 + c2 * 16)
    idx3 = _perm_idx(r * 4096 + c0 + c1 * 16 + c2 * 256)
    cat = jnp.concatenate([idx2, idx3], axis=1)
    o_ref[...] = cat.reshape(o_ref.shape)


def _idx(codes):
    ct = jnp.transpose(codes, (0, 2, 1, 3))
    return pl.pallas_call(
        _idx_body,
        grid=(_B, 2),
        in_specs=[pl.BlockSpec((1, _T, _S // 2, _R), lambda i, j: (i, 0, j, 0))],
        out_specs=pl.BlockSpec((4096, 128), lambda i, j: (i * 2 + j, 0)),
        out_shape=jax.ShapeDtypeStruct((_PAIRS * 4, 128), jnp.int32),
    )(ct)


def _assemble_body(x_ref, o_ref):
    for ct in range(64):
        o_ref[0, :, ct * 128:(ct + 1) * 128] = x_ref[:, ct, :]


def _assemble(out2d):
    x = out2d.reshape(_PAIRS, 64, 128)
    return pl.pallas_call(
        _assemble_body,
        grid=(_B, 8),
        in_specs=[pl.BlockSpec((256, 64, 128), lambda i, j: (i * 8 + j, 0, 0))],
        out_specs=pl.BlockSpec((1, 256, 8192), lambda i, j: (i, j, 0)),
        out_shape=jax.ShapeDtypeStruct((_B, _S, 2 * _R * _MEM), jnp.float32),
    )(x)


def _sc_gather(idx2d, w2s, w3s):
    mesh = plsc.VectorSubcoreMesh(
        core_axis_name="c", subcore_axis_name="s",
        num_cores=_NC, num_subcores=_NS)

    @functools.partial(
        pl.kernel,
        out_type=jax.ShapeDtypeStruct((_PAIRS * 512, _MEM), jnp.float32),
        mesh=mesh,
        scratch_types=[
            pltpu.VMEM((_NBUF, 4 * _G, 128), jnp.int32),
            pltpu.VMEM((_NBUF, 4 * _G * 128, _MEM), jnp.float32),
            pltpu.SemaphoreType.DMA((_NBUF,)),
            pltpu.SemaphoreType.DMA((_NBUF,)),
            pltpu.SemaphoreType.DMA((_NBUF,)),
        ],
        compiler_params=pltpu.CompilerParams(use_tc_tiling_on_sc=False),
    )
    def k(idx_hbm, w2_hbm, w3_hbm, out_hbm, idx_v, rows_v, isem, gsem, ssem):
        wid = lax.axis_index("s") * _NC + lax.axis_index("c")
        g0 = wid * _NGW

        def idx_cp(slot, g):
            return pltpu.make_async_copy(
                idx_hbm.at[pl.ds((g0 + g) * (4 * _G), 4 * _G)],
                idx_v.at[slot], isem.at[slot])

        def gath_cps(slot):
            cps = []
            for q in range(_G):
                for j in range(4):
                    tbl = w2_hbm if j < 2 else w3_hbm
                    cps.append(pltpu.make_async_copy(
                        tbl.at[idx_v.at[slot, 4 * q + j]],
                        rows_v.at[slot, pl.ds((4 * q + j) * 128, 128)],
                        gsem.at[slot]))
            return cps

        def store_cp(slot, g):
            return pltpu.make_async_copy(
                rows_v.at[slot],
                out_hbm.at[pl.ds((g0 + g) * (512 * _G), 512 * _G)], ssem.at[slot])

        idx_cp(0, 0).start()
        idx_cp(1, 1).start()
        idx_cp(0, 0).wait()
        for cp in gath_cps(0):
            cp.start()
        for cp in gath_cps(0):
            cp.wait()
        store_cp(0, 0).start()
        idx_cp(1, 1).wait()
        for cp in gath_cps(1):
            cp.start()
        idx_cp(0, 2).start()

        @pl.loop(2, _NGW - 2, step=_NBUF)
        def _outer(go):
            for b in range(_NBUF):
                g = go + b
                prev = 1 - b
                for cp in gath_cps(prev):
                    cp.wait()
                store_cp(prev, g - 1).start()
                store_cp(b, g - 2).wait()
                idx_cp(b, g).wait()
                for cp in gath_cps(b):
                    cp.start()
                idx_cp(prev, g + 1).start()

        for cp in gath_cps(1):
            cp.wait()
        store_cp(1, _NGW - 3).start()
        store_cp(0, _NGW - 4).wait()
        idx_cp(0, _NGW - 2).wait()
        for cp in gath_cps(0):
            cp.start()
        idx_cp(1, _NGW - 1).start()

        for cp in gath_cps(0):
            cp.wait()
        store_cp(0, _NGW - 2).start()
        store_cp(1, _NGW - 3).wait()
        idx_cp(1, _NGW - 1).wait()
        for cp in gath_cps(1):
            cp.start()

        for cp in gath_cps(1):
            cp.wait()
        store_cp(1, _NGW - 1).start()
        store_cp(0, _NGW - 2).wait()
        store_cp(1, _NGW - 1).wait()

    return k(idx2d, w2s, w3s)


def kernel(route_codes_bstr, W_ngram_2, W_ngram_3):
    idx2d = _idx(route_codes_bstr)
    w2s = _presum(W_ngram_2, _V2).reshape(_V2, _MEM)
    w3s = _presum(W_ngram_3, _V3).reshape(_V3, _MEM)
    out2d = _sc_gather(idx2d, w2s, w3s)
    return _assemble(out2d)


# speedup vs baseline: 2.0729x; 1.1901x over previous
import functools

import jax
import jax.numpy as jnp
from jax import lax
from jax.experimental import pallas as pl
from jax.experimental.pallas import tpu as pltpu
from jax.experimental.pallas import tpu_sc as plsc

_B, _S, _T, _R = 4, 2048, 3, 256
_ALPHA, _MEM = 16, 16
_PAIRS = _B * _S
_V2 = _R * _ALPHA ** 2
_V3 = _R * _ALPHA ** 3

_NC, _NS = 2, 16
_NW = _NC * _NS
_PPW = _PAIRS // _NW

_G = 8
_NBUF = 2
_NGW = _PPW // _G


_CH = 16384
_C8 = _CH // 8


def _presum_body(w_ref, o_ref):
    x = w_ref[0] + w_ref[1]
    r = lax.broadcasted_iota(jnp.int32, (128, 128), 0)
    c = lax.broadcasted_iota(jnp.int32, (128, 128), 1)
    eye = (r == c).astype(jnp.float32)
    xs = jnp.concatenate(
        [x[:, i * _C8:(i + 1) * _C8] for i in range(8)], axis=0)
    o_ref[...] = lax.dot_general(
        xs, eye, dimension_numbers=(((0,), (0,)), ((), ())),
        preferred_element_type=jnp.float32)


def _presum(w, v):
    wt = jnp.transpose(w, (0, 2, 1))
    return pl.pallas_call(
        _presum_body,
        grid=(v // _CH,),
        in_specs=[pl.BlockSpec((2, 16, _CH), lambda i: (0, 0, i))],
        out_specs=pl.BlockSpec((_C8, 128), lambda i: (i, 0)),
        out_shape=jax.ShapeDtypeStruct((v // 8, 128), jnp.float32),
    )(wt)


def _perm_idx(t):
    return (t & ~(_CH - 1)) | ((t & (_C8 - 1)) << 3) | ((t >> 11) & 7)


def _idx_body(c_ref, o2_ref, o3_ref):
    x = c_ref[0]
    c0 = x[0]
    c1 = x[1]
    c2 = x[2]
    r = lax.broadcasted_iota(jnp.int32, c0.shape, 1)
    idx2 = _perm_idx(r * 256 + c1 + c2 * 16)
    idx3 = _perm_idx(r * 4096 + c0 + c1 * 16 + c2 * 256)
    o2_ref[...] = idx2.reshape(o2_ref.shape)
    o3_ref[...] = idx3.reshape(o3_ref.shape)


def _idx(codes):
    ct = jnp.transpose(codes, (0, 2, 1, 3))
    return pl.pallas_call(
        _idx_body,
        grid=(_B, 2),
        in_specs=[pl.BlockSpec((1, _T, _S // 2, _R), lambda i, j: (i, 0, j, 0))],
        out_specs=[pl.BlockSpec((2048, 128), lambda i, j: (i * 2 + j, 0))] * 2,
        out_shape=[jax.ShapeDtypeStruct((_PAIRS * 2, 128), jnp.int32)] * 2,
    )(ct)


def _assemble_a_body(x_ref, o_ref):
    for ct in range(32):
        o_ref[0, :, ct * 128:(ct + 1) * 128] = x_ref[:, ct, :]


def _assemble_b_body(p_ref, x_ref, o_ref):
    del p_ref
    for ct in range(32):
        o_ref[0, :, ct * 128:(ct + 1) * 128] = x_ref[:, ct, :]


def _assemble_a(outh):
    x = outh.reshape(_PAIRS, 32, 128)
    return pl.pallas_call(
        _assemble_a_body,
        grid=(_B, 8),
        in_specs=[pl.BlockSpec((256, 32, 128), lambda i, j: (i * 8 + j, 0, 0))],
        out_specs=pl.BlockSpec((1, 256, 4096), lambda i, j: (i, j, 0)),
        out_shape=jax.ShapeDtypeStruct((_B, _S, 2 * _R * _MEM), jnp.float32),
    )(x)


def _assemble_b(half, outh):
    x = outh.reshape(_PAIRS, 32, 128)
    return pl.pallas_call(
        _assemble_b_body,
        grid=(_B, 8),
        in_specs=[
            pl.BlockSpec((1, 8, 4096), lambda i, j: (0, 0, 0)),
            pl.BlockSpec((256, 32, 128), lambda i, j: (i * 8 + j, 0, 0)),
        ],
        out_specs=pl.BlockSpec((1, 256, 4096), lambda i, j: (i, j, 1)),
        out_shape=jax.ShapeDtypeStruct((_B, _S, 2 * _R * _MEM), jnp.float32),
        input_output_aliases={0: 0},
    )(half, x)


def _sc_gather_one(idx2d, ws):
    mesh = plsc.VectorSubcoreMesh(
        core_axis_name="c", subcore_axis_name="s",
        num_cores=_NC, num_subcores=_NS)

    @functools.partial(
        pl.kernel,
        out_type=jax.ShapeDtypeStruct((_PAIRS * 256, _MEM), jnp.float32),
        mesh=mesh,
        scratch_types=[
            pltpu.VMEM((_NBUF, 2 * _G, 128), jnp.int32),
            pltpu.VMEM((_NBUF, 2 * _G * 128, _MEM), jnp.float32),
            pltpu.SemaphoreType.DMA((_NBUF,)),
            pltpu.SemaphoreType.DMA((_NBUF,)),
            pltpu.SemaphoreType.DMA((_NBUF,)),
        ],
        compiler_params=pltpu.CompilerParams(use_tc_tiling_on_sc=False),
    )
    def k(idx_hbm, w_hbm, out_hbm, idx_v, rows_v, isem, gsem, ssem):
        wid = lax.axis_index("s") * _NC + lax.axis_index("c")
        g0 = wid * _NGW

        def idx_cp(slot, g):
            return pltpu.make_async_copy(
                idx_hbm.at[pl.ds((g0 + g) * (2 * _G), 2 * _G)],
                idx_v.at[slot], isem.at[slot])

        def gath_cps(slot):
            cps = []
            for q in range(2 * _G):
                cps.append(pltpu.make_async_copy(
                    w_hbm.at[idx_v.at[slot, q]],
                    rows_v.at[slot, pl.ds(q * 128, 128)],
                    gsem.at[slot]))
            return cps

        def store_cp(slot, g):
            return pltpu.make_async_copy(
                rows_v.at[slot],
                out_hbm.at[pl.ds((g0 + g) * (256 * _G), 256 * _G)], ssem.at[slot])

        idx_cp(0, 0).start()
        idx_cp(1, 1).start()
        idx_cp(0, 0).wait()
        for cp in gath_cps(0):
            cp.start()
        for cp in gath_cps(0):
            cp.wait()
        store_cp(0, 0).start()
        idx_cp(1, 1).wait()
        for cp in gath_cps(1):
            cp.start()
        idx_cp(0, 2).start()

        @pl.loop(2, _NGW - 2, step=_NBUF)
        def _outer(go):
            for b in range(_NBUF):
                g = go + b
                prev = 1 - b
                for cp in gath_cps(prev):
                    cp.wait()
                store_cp(prev, g - 1).start()
                store_cp(b, g - 2).wait()
                idx_cp(b, g).wait()
                for cp in gath_cps(b):
                    cp.start()
                idx_cp(prev, g + 1).start()

        for cp in gath_cps(1):
            cp.wait()
        store_cp(1, _NGW - 3).start()
        store_cp(0, _NGW - 4).wait()
        idx_cp(0, _NGW - 2).wait()
        for cp in gath_cps(0):
            cp.start()
        idx_cp(1, _NGW - 1).start()

        for cp in gath_cps(0):
            cp.wait()
        store_cp(0, _NGW - 2).start()
        store_cp(1, _NGW - 3).wait()
        idx_cp(1, _NGW - 1).wait()
        for cp in gath_cps(1):
            cp.start()

        for cp in gath_cps(1):
            cp.wait()
        store_cp(1, _NGW - 1).start()
        store_cp(0, _NGW - 2).wait()
        store_cp(1, _NGW - 1).wait()

    return k(idx2d, ws)


def kernel(route_codes_bstr, W_ngram_2, W_ngram_3):
    i2, i3 = _idx(route_codes_bstr)
    w2s = _presum(W_ngram_2, _V2).reshape(_V2, _MEM)
    w3s = _presum(W_ngram_3, _V3).reshape(_V3, _MEM)
    outa = _sc_gather_one(i2, w2s)
    outb = _sc_gather_one(i3, w3s)
    half = _assemble_a(outa)
    return _assemble_b(half, outb)
